# baseline (speedup 1.0000x reference)
import functools

import jax
import jax.numpy as jnp
from jax import lax
from jax.experimental import pallas as pl
from jax.experimental.pallas import tpu as pltpu
from jax.experimental.pallas import tpu_sc as plsc

F = 26
B = 4096
H = 20
V = 100000
D = 32
L = 16

NW = 32
SLOTS_PER_TILE = (B * F) // NW
C = 832
CHUNKS = SLOTS_PER_TILE // C


def _make_sc_kernel():
    info = plsc.get_sparse_core_info()
    nc = info.num_cores
    mesh = plsc.VectorSubcoreMesh(core_axis_name="c", subcore_axis_name="s")

    @functools.partial(
        pl.kernel,
        mesh=mesh,
        compiler_params=pltpu.CompilerParams(
            use_tc_tiling_on_sc=False, needs_layout_passes=False
        ),
        out_type=jax.ShapeDtypeStruct((B * F, D), jnp.float32),
        scratch_types=[
            pltpu.VMEM((C * H,), jnp.int32),
            pltpu.VMEM((H, C), jnp.int32),
            pltpu.VMEM((C,), jnp.int32),
            pltpu.VMEM((C, D), jnp.float32),
            pltpu.SemaphoreType.DMA,
        ],
    )
    def k(table_hbm, idx_hbm, zeros_hbm, out_hbm, raw_v, idxl_v, dst_v, acc_v, sem):
        wid = lax.axis_index("s") * nc + lax.axis_index("c")
        tile_base = wid * SLOTS_PER_TILE
        iota = lax.iota(jnp.int32, L)

        def chunk_body(ci, carry):
            slot_base = tile_base + ci * C
            pltpu.sync_copy(idx_hbm.at[pl.ds(slot_base * H, C * H)], raw_v)

            def build_body(j, c2):
                s = slot_base + j * L + iota
                f = lax.shift_right_logical(s, 12)
                b = lax.bitwise_and(s, B - 1)
                dst_v[pl.ds(j * L, L)] = b * F + f
                off = f * V
                p0 = (j * L + iota) * H
                for l in range(H):
                    idxl_v[l, pl.ds(j * L, L)] = plsc.load_gather(raw_v, [p0 + l]) + off
                return c2

            lax.fori_loop(0, C // L, build_body, 0)

            pltpu.sync_copy(zeros_hbm, acc_v)
            for l in range(H):
                pltpu.async_copy(table_hbm.at[idxl_v.at[l]], acc_v, sem, add=True)
            for l in range(H):
                pltpu.make_async_copy(table_hbm.at[idxl_v.at[l]], acc_v, sem).wait()

            pltpu.sync_copy(acc_v, out_hbm.at[dst_v])
            return carry

        lax.fori_loop(0, CHUNKS, chunk_body, 0)

    return k


_sc_kernel = _make_sc_kernel()


@jax.jit
def kernel(inputs, tables):
    idx_flat = inputs.astype(jnp.int32).reshape(F * B * H)
    tables_flat = tables.reshape(F * V, D)
    zeros = jnp.zeros((C, D), jnp.float32)
    out = _sc_kernel(tables_flat, idx_flat, zeros)
    return out.reshape(B, F, D)


# speedup vs baseline: 1.3911x; 1.3911x over previous
import functools

import jax
import jax.numpy as jnp
from jax import lax
from jax.experimental import pallas as pl
from jax.experimental.pallas import tpu as pltpu
from jax.experimental.pallas import tpu_sc as plsc

F = 26
B = 4096
H = 20
V = 100000
D = 32
L = 16

NW = 32
PAIRS_PER_TILE = (F * D) // NW


def _make_sc_kernel():
    info = plsc.get_sparse_core_info()
    nc = info.num_cores
    mesh = plsc.VectorSubcoreMesh(core_axis_name="c", subcore_axis_name="s")

    @functools.partial(
        pl.kernel,
        mesh=mesh,
        compiler_params=pltpu.CompilerParams(needs_layout_passes=False),
        out_type=jax.ShapeDtypeStruct((F, D, B), jnp.float32),
        scratch_types=[
            pltpu.VMEM((V,), jnp.float32),
            pltpu.VMEM((B,), jnp.int32),
            pltpu.VMEM((B,), jnp.int32),
            pltpu.VMEM((B,), jnp.float32),
            pltpu.SemaphoreType.DMA,
        ],
    )
    def k(tab_hbm, idx_hbm, out_hbm, slab_v, idx_a, idx_b, acc_v, sem):
        wid = lax.axis_index("s") * nc + lax.axis_index("c")

        def pair_body(i, carry):
            p = wid * PAIRS_PER_TILE + i
            f = lax.shift_right_logical(p, 5)
            d = lax.bitwise_and(p, D - 1)
            pltpu.sync_copy(tab_hbm.at[f, d], slab_v)
            pltpu.sync_copy(idx_hbm.at[f, 0], idx_a)
            bufs = (idx_a, idx_b)
            for l in range(H):
                cur = bufs[l % 2]
                nxt = bufs[(l + 1) % 2]
                if l + 1 < H:
                    cp = pltpu.async_copy(idx_hbm.at[f, l + 1], nxt, sem)

                if l == 0:
                    def j_body(j, c2):
                        iv = cur[pl.ds(j * L, L)]
                        acc_v[pl.ds(j * L, L)] = plsc.load_gather(slab_v, [iv])
                        return c2
                else:
                    def j_body(j, c2):
                        iv = cur[pl.ds(j * L, L)]
                        g = plsc.load_gather(slab_v, [iv])
                        plsc.addupdate(acc_v.at[pl.ds(j * L, L)], g)
                        return c2

                lax.fori_loop(0, B // L, j_body, 0)
                if l + 1 < H:
                    cp.wait()
            pltpu.sync_copy(acc_v, out_hbm.at[f, d])
            return carry

        lax.fori_loop(0, PAIRS_PER_TILE, pair_body, 0)

    return k


_sc_kernel = _make_sc_kernel()


@jax.jit
def kernel(inputs, tables):
    idx_t = jnp.transpose(inputs.astype(jnp.int32), (0, 2, 1))
    tab_t = jnp.transpose(tables, (0, 2, 1))
    out = _sc_kernel(tab_t, idx_t)
    return jnp.transpose(out, (2, 0, 1))


# speedup vs baseline: 2.4299x; 1.7467x over previous
import functools

import jax
import jax.numpy as jnp
from jax import lax
from jax.experimental import pallas as pl
from jax.experimental.pallas import tpu as pltpu
from jax.experimental.pallas import tpu_sc as plsc

F = 26
B = 4096
H = 20
V = 100000
D = 32
L = 16

NW = 32
PAIRS_PER_TILE = (F * D) // NW


def _make_sc_kernel():
    info = plsc.get_sparse_core_info()
    nc = info.num_cores
    mesh = plsc.VectorSubcoreMesh(core_axis_name="c", subcore_axis_name="s")

    @functools.partial(
        pl.kernel,
        mesh=mesh,
        compiler_params=pltpu.CompilerParams(needs_layout_passes=False),
        out_type=jax.ShapeDtypeStruct((F, D, B), jnp.float32),
        scratch_types=[
            pltpu.VMEM((V,), jnp.float32),
            pltpu.VMEM((B,), jnp.int32),
            pltpu.VMEM((B,), jnp.int32),
            pltpu.VMEM((B,), jnp.float32),
            pltpu.SemaphoreType.DMA,
        ],
    )
    def k(tab_hbm, idx_hbm, out_hbm, slab_v, idx_a, idx_b, acc_v, sem):
        wid = lax.axis_index("s") * nc + lax.axis_index("c")

        def pair_body(i, carry):
            p = wid * PAIRS_PER_TILE + i
            f = lax.shift_right_logical(p, 5)
            d = lax.bitwise_and(p, D - 1)
            pltpu.sync_copy(tab_hbm.at[f, d], slab_v)
            pltpu.sync_copy(idx_hbm.at[f, 0], idx_a)
            bufs = (idx_a, idx_b)
            for l in range(H):
                cur = bufs[l % 2]
                nxt = bufs[(l + 1) % 2]
                if l + 1 < H:
                    cp = pltpu.async_copy(idx_hbm.at[f, l + 1], nxt, sem)

                first = l == 0

                @plsc.parallel_loop(0, B // L, unroll=8)
                def j_body(j):
                    iv = cur[pl.ds(j * L, L)]
                    g = plsc.load_gather(slab_v, [iv])
                    if first:
                        acc_v[pl.ds(j * L, L)] = g
                    else:
                        plsc.addupdate(acc_v.at[pl.ds(j * L, L)], g)
                if l + 1 < H:
                    cp.wait()
            pltpu.sync_copy(acc_v, out_hbm.at[f, d])
            return carry

        lax.fori_loop(0, PAIRS_PER_TILE, pair_body, 0)

    return k


_sc_kernel = _make_sc_kernel()


@jax.jit
def kernel(inputs, tables):
    idx_t = jnp.transpose(inputs.astype(jnp.int32), (0, 2, 1))
    tab_t = jnp.transpose(tables, (0, 2, 1))
    out = _sc_kernel(tab_t, idx_t)
    return jnp.transpose(out, (2, 0, 1))
